# baseline (speedup 1.0000x reference)
import functools
import jax
import jax.numpy as jnp
from jax import lax
from jax.experimental import pallas as pl
from jax.experimental.pallas import tpu as pltpu
from jax.experimental.pallas import tpu_sc as plsc

_N_PATCHES = 16
_CH = 8
_NBUF = 4
_SC_ROWS = 4096
_TC_BLK = 1024


@functools.lru_cache(maxsize=None)
def _make_sc_kernel(Rsc, D, S, row_base):
    info = plsc.get_sparse_core_info()
    NC, NS, L = info.num_cores, info.num_subcores, info.num_lanes
    NW = NC * NS
    rows_w = Rsc // NW
    rpp = S // _N_PATCHES
    nchunks = rows_w // _CH
    cols = D // L

    mesh = plsc.VectorSubcoreMesh(core_axis_name="c", subcore_axis_name="s")

    @functools.partial(
        pl.kernel,
        out_type=jax.ShapeDtypeStruct((Rsc, D), jnp.float32),

        mesh=mesh,
        scratch_types=[
            pltpu.VMEM((_N_PATCHES, D), jnp.float32),
            [pltpu.VMEM((_CH, D), jnp.float32)] * _NBUF,
            [pltpu.SemaphoreType.DMA] * _NBUF,
            [pltpu.SemaphoreType.DMA] * _NBUF,
        ],
        compiler_params=pltpu.CompilerParams(use_tc_tiling_on_sc=True),
    )
    def sc_k(vf_hbm, w_hbm, out_hbm, w_buf, bufs, sins, souts):
        cid = lax.axis_index("c")
        sid = lax.axis_index("s")
        wid = sid * NC + cid
        row0 = wid * rows_w
        pltpu.sync_copy(w_hbm.at[pl.ds(0, _N_PATCHES)], w_buf)

        def issue_gather(g, b):
            rs = row_base + row0 + g * _CH
            pltpu.async_copy(vf_hbm.at[pl.ds(rs, _CH)], bufs[b], sins[b])

        def issue_scatter(g, b):
            rs = row0 + g * _CH
            pltpu.async_copy(bufs[b], out_hbm.at[pl.ds(rs, _CH)], souts[b])

        def wait_in(b):
            pltpu.make_async_copy(vf_hbm.at[pl.ds(0, _CH)], bufs[b], sins[b]).wait()

        def wait_out(b):
            pltpu.make_async_copy(bufs[b], out_hbm.at[pl.ds(0, _CH)], souts[b]).wait()

        issue_gather(0, 0)
        issue_gather(1, 1)

        def quad(t, carry):
            for b in range(_NBUF):
                j = t * _NBUF + b
                wait_in(b)

                p = ((row_base + row0 + j * _CH) % S) // rpp

                @plsc.parallel_loop(0, cols, 1, unroll=4)
                def col(c):
                    wv = w_buf[p, pl.ds(c * L, L)]
                    for r in range(_CH):
                        plsc.addupdate(bufs[b].at[r, pl.ds(c * L, L)], wv)

                issue_scatter(j, b)

                bn = (b + 2) % _NBUF

                @pl.when(j >= 2)
                def _():
                    wait_out(bn)

                @pl.when(j + 2 < nchunks)
                def _():
                    issue_gather(j + 2, bn)

            return carry

        lax.fori_loop(0, nchunks // _NBUF, quad, 0)
        wait_out((nchunks - 2) % _NBUF)
        wait_out((nchunks - 1) % _NBUF)

    return sc_k


def _tc_body(vf_ref, w_ref, out_ref):
    blk, d = vf_ref.shape
    ppb = w_ref.shape[0]
    rpp = blk // ppb
    x = vf_ref[...].reshape(ppb, rpp, d) + w_ref[...]
    out_ref[...] = x.reshape(blk, d)


def _tc_part(vf, W, row_base, S):
    R, D = vf.shape
    Rtc = R - row_base
    rpp = S // _N_PATCHES
    ppb = _TC_BLK // rpp
    w3 = W.reshape(W.shape[0], 1, D)
    nblk = Rtc // _TC_BLK
    wblocks = _N_PATCHES // ppb
    base_blk = row_base // _TC_BLK

    def w_index(k):
        return ((base_blk + k) % (S // _TC_BLK)) % wblocks, 0, 0

    return pl.pallas_call(
        _tc_body,
        grid=(nblk,),
        in_specs=[
            pl.BlockSpec((_TC_BLK, D), lambda k: (k + base_blk, 0)),
            pl.BlockSpec((ppb, 1, D), w_index),
        ],
        out_specs=pl.BlockSpec((_TC_BLK, D), lambda k: (k, 0)),
        out_shape=jax.ShapeDtypeStruct((Rtc, D), vf.dtype),
    )(vf, w3)


def kernel(vision_features, W):
    B, S, D = vision_features.shape
    R = B * S
    vf = vision_features.reshape(R, D)
    sc_k = _make_sc_kernel(_SC_ROWS, D, S, 0)
    out_sc = sc_k(vf, W)
    out_tc = _tc_part(vf, W, _SC_ROWS, S)
    out = jnp.concatenate([out_sc, out_tc], axis=0)
    return out.reshape(B, S, D)


# speedup vs baseline: 1.6204x; 1.6204x over previous
import functools
import jax
import jax.numpy as jnp
from jax import lax
from jax.experimental import pallas as pl
from jax.experimental.pallas import tpu as pltpu
from jax.experimental.pallas import tpu_sc as plsc

_N_PATCHES = 16
_CH = 16
_NBUF = 2


@functools.lru_cache(maxsize=None)
def _make_sc_kernel(R, D, S):
    info = plsc.get_sparse_core_info()
    NC, NS, L = info.num_cores, info.num_subcores, info.num_lanes
    NW = NC * NS
    rows_w = R // NW
    rpp = S // _N_PATCHES
    ppw = rows_w // rpp
    wpb = S // rows_w
    nchunks = rows_w // _CH
    cpp = rpp // _CH
    cols = D // L

    mesh = plsc.VectorSubcoreMesh(core_axis_name="c", subcore_axis_name="s")

    @functools.partial(
        pl.kernel,
        out_type=jax.ShapeDtypeStruct((R, D), jnp.float32),
        mesh=mesh,
        scratch_types=[
            pltpu.VMEM((_N_PATCHES, D), jnp.float32),
            [pltpu.VMEM((_CH, D), jnp.float32)] * _NBUF,
            [pltpu.SemaphoreType.DMA] * _NBUF,
            [pltpu.SemaphoreType.DMA] * _NBUF,
        ],
        compiler_params=pltpu.CompilerParams(use_tc_tiling_on_sc=True),
    )
    def sc_k(vf_hbm, w_hbm, out_hbm, w_buf, bufs, sins, souts):
        cid = lax.axis_index("c")
        sid = lax.axis_index("s")
        wid = sid * NC + cid
        row0 = wid * rows_w
        p0 = (wid % wpb) * ppw
        pltpu.sync_copy(w_hbm.at[pl.ds(0, _N_PATCHES)], w_buf)

        def issue_gather(g, b):
            rs = row0 + g * _CH
            pltpu.async_copy(vf_hbm.at[pl.ds(rs, _CH)], bufs[b], sins[b])

        def issue_scatter(g, b):
            rs = row0 + g * _CH
            pltpu.async_copy(bufs[b], out_hbm.at[pl.ds(rs, _CH)], souts[b])

        def wait_in(b):
            pltpu.make_async_copy(vf_hbm.at[pl.ds(0, _CH)], bufs[b], sins[b]).wait()

        def wait_out(b):
            pltpu.make_async_copy(bufs[b], out_hbm.at[pl.ds(0, _CH)], souts[b]).wait()

        issue_gather(0, 0)
        issue_gather(1, 1)

        def quad(t, carry):
            for b in range(_NBUF):
                j = t * _NBUF + b
                wait_in(b)

                p = p0 + j // cpp

                @plsc.parallel_loop(0, cols, 1, unroll=4)
                def col(c):
                    wv = w_buf[p, pl.ds(c * L, L)]
                    for r in range(_CH):
                        plsc.addupdate(bufs[b].at[r, pl.ds(c * L, L)], wv)
                issue_scatter(j, b)

                bn = (b + 2) % _NBUF

                @pl.when(j >= 2)
                def _():
                    wait_out(bn)

                @pl.when(j + 2 < nchunks)
                def _():
                    issue_gather(j + 2, bn)

            return carry

        lax.fori_loop(0, nchunks // _NBUF, quad, 0)
        wait_out((nchunks - 2) % _NBUF)
        wait_out((nchunks - 1) % _NBUF)

    return sc_k


def kernel(vision_features, W):
    B, S, D = vision_features.shape
    R = B * S
    vf = vision_features.reshape(R, D)
    sc_k = _make_sc_kernel(R, D, S)
    out = sc_k(vf, W)
    return out.reshape(B, S, D)
